# Initial kernel scaffold; baseline (speedup 1.0000x reference)
import jax
import jax.numpy as jnp
from jax.experimental import pallas as pl


def kernel(x, codebook, W1, b1, W2, b2, W3, b3, dense_index, sparse_index):
    raise NotImplementedError("write your pallas kernel here")


# speedup vs baseline: 4.8931x; 4.8931x over previous
import functools

import jax
import jax.numpy as jnp
from jax import lax
from jax.experimental import pallas as pl
from jax.experimental.pallas import tpu as pltpu
from jax.experimental.pallas import tpu_sc as plsc

NUM_DENSE = 13
NUM_SPARSE = 26
VOCAB = 1000
EMBED = 64
BATCH = 4096

NC = 2
NS = 16
NW = NC * NS
ROWS_PER_W = BATCH // NW
CHUNK = 128
N_IDX = BATCH * NUM_SPARSE
IDX_PER_W = N_IDX // NW
N_CHUNKS = IDX_PER_W // CHUNK

@functools.cache
def _build_sc_gather():
    mesh = plsc.VectorSubcoreMesh(
        core_axis_name="c", subcore_axis_name="s",
        num_cores=NC, num_subcores=NS)

    @functools.partial(
        pl.kernel,
        out_type=jax.ShapeDtypeStruct((N_IDX, EMBED), jnp.float32),
        mesh=mesh,
        scratch_types=[
            pltpu.VMEM((N_CHUNKS, CHUNK), jnp.int32),
            pltpu.VMEM((CHUNK, EMBED), jnp.float32),
            pltpu.VMEM((CHUNK, EMBED), jnp.float32),
            pltpu.SemaphoreType.DMA,
            pltpu.SemaphoreType.DMA,
        ],
        compiler_params=pltpu.CompilerParams(use_tc_tiling_on_sc=False),
    )
    def _sc_gather(table_hbm, idx_hbm, out_hbm, idx_v, buf0, buf1, sem0, sem1):
        wid = lax.axis_index("s") * NC + lax.axis_index("c")
        pltpu.sync_copy(idx_hbm.at[wid], idx_v)
        base = wid * IDX_PER_W

        def step(j, carry):
            j0 = 2 * j
            j1 = j0 + 1
            cp0 = pltpu.async_copy(table_hbm.at[idx_v.at[j0]], buf0, sem0)
            cp1 = pltpu.async_copy(table_hbm.at[idx_v.at[j1]], buf1, sem1)
            cp0.wait()
            pltpu.sync_copy(buf0, out_hbm.at[pl.ds(base + j0 * CHUNK, CHUNK)])
            cp1.wait()
            pltpu.sync_copy(buf1, out_hbm.at[pl.ds(base + j1 * CHUNK, CHUNK)])
            return carry

        lax.fori_loop(0, N_CHUNKS // 2, step, 0)

    return _sc_gather


BLK = 512
D_SP = NUM_SPARSE * EMBED


def _mlp_body(x_ref, sf_ref, w1d_ref, w1s_ref, b1_ref, w2_ref, b2_ref,
              w3_ref, b3_ref, out_ref):
    xd = x_ref[:, :NUM_DENSE]
    h1 = jnp.dot(sf_ref[...], w1s_ref[...], preferred_element_type=jnp.float32)
    h1 = h1 + jnp.dot(xd, w1d_ref[...], preferred_element_type=jnp.float32)
    h1 = jnp.maximum(h1 + b1_ref[...], 0.0)
    h2 = jnp.dot(h1, w2_ref[...], preferred_element_type=jnp.float32)
    h2 = jnp.maximum(h2 + b2_ref[...], 0.0)
    out_ref[...] = (jnp.dot(h2, w3_ref[...], preferred_element_type=jnp.float32)
                    + b3_ref[...])


def _mlp(x, sf, w1d, w1s, b1, w2, b2, w3, b3):
    grid = (BATCH // BLK,)
    const = lambda i: (0, 0)
    return pl.pallas_call(
        _mlp_body,
        grid=grid,
        in_specs=[
            pl.BlockSpec((BLK, x.shape[1]), lambda i: (i, 0)),
            pl.BlockSpec((BLK, D_SP), lambda i: (i, 0)),
            pl.BlockSpec(w1d.shape, const),
            pl.BlockSpec(w1s.shape, const),
            pl.BlockSpec(b1.shape, const),
            pl.BlockSpec(w2.shape, const),
            pl.BlockSpec(b2.shape, const),
            pl.BlockSpec(w3.shape, const),
            pl.BlockSpec(b3.shape, const),
        ],
        out_specs=pl.BlockSpec((BLK, 2), lambda i: (i, 0)),
        out_shape=jax.ShapeDtypeStruct((BATCH, 2), jnp.float32),
        compiler_params=pltpu.CompilerParams(
            dimension_semantics=("parallel",),
        ),
    )(x, sf, w1d, w1s, b1, w2, b2, w3, b3)


def kernel(x, codebook, W1, b1, W2, b2, W3, b3, dense_index, sparse_index):
    x_sp = jnp.take(x, sparse_index, axis=1).astype(jnp.int32)
    x_sp = jnp.where(x_sp == -1, VOCAB - 1, x_sp)
    idx = x_sp + (jnp.arange(NUM_SPARSE, dtype=jnp.int32) * VOCAB)[None, :]
    idx = idx.reshape(NW, N_CHUNKS, CHUNK)
    table = codebook.reshape(NUM_SPARSE * VOCAB, EMBED)

    sf = _build_sc_gather()(table, idx)
    sf = sf.reshape(BATCH, D_SP)

    w1d = W1[:NUM_DENSE]
    w1s = W1[NUM_DENSE:]
    return _mlp(x, sf, w1d, w1s, b1.reshape(1, -1), W2, b2.reshape(1, -1),
                W3, b3.reshape(1, -1))
